# baseline (speedup 1.0000x reference)
import functools

import jax
import jax.numpy as jnp
from jax import lax
from jax.experimental import pallas as pl
from jax.experimental.pallas import tpu as pltpu
from jax.experimental.pallas import tpu_sc as plsc

_B = 16384
_D = 16
_N = 1000000
_NC = 2
_NS = 16
_NW = _NC * _NS
_BPW = _B // _NW
_RB = 16


def _body(user_hbm, item_hbm, utab_hbm, itab_hbm, out_hbm,
          idx_u, idx_i, cols_u, cols_i, out_v, sem):
    wid = lax.axis_index("s") * _NC + lax.axis_index("c")
    base = wid * _BPW

    pltpu.sync_copy(user_hbm.at[pl.ds(base, _BPW)], idx_u)
    pltpu.sync_copy(item_hbm.at[pl.ds(base, _BPW)], idx_i)

    cps = []
    for d in range(_D):
        cps.append(pltpu.async_copy(utab_hbm.at[d].at[idx_u],
                                    cols_u.at[d], sem))
        cps.append(pltpu.async_copy(itab_hbm.at[d].at[idx_i],
                                    cols_i.at[d], sem))
    for cp in cps:
        cp.wait()

    def block(b, carry):
        i0 = b * _RB
        acc = jnp.zeros((_RB,), jnp.float32)
        for d in range(_D):
            acc = acc + cols_u[d, pl.ds(i0, _RB)] * cols_i[d, pl.ds(i0, _RB)]
        out_v[pl.ds(i0, _RB)] = 1.0 / (1.0 + jnp.exp(-acc))
        return carry

    lax.fori_loop(0, _BPW // _RB, block, 0)

    pltpu.sync_copy(out_v, out_hbm.at[pl.ds(base, _BPW)])


def kernel(user, item, user_table, item_table):
    utab_t = user_table.T
    itab_t = item_table.T
    mesh = plsc.VectorSubcoreMesh(core_axis_name="c", subcore_axis_name="s")
    f = functools.partial(
        pl.kernel,
        out_type=jax.ShapeDtypeStruct((_B,), jnp.float32),
        mesh=mesh,
        scratch_types=[
            pltpu.VMEM((_BPW,), jnp.int32),
            pltpu.VMEM((_BPW,), jnp.int32),
            pltpu.VMEM((_D, _BPW), jnp.float32),
            pltpu.VMEM((_D, _BPW), jnp.float32),
            pltpu.VMEM((_BPW,), jnp.float32),
            pltpu.SemaphoreType.DMA,
        ],
        compiler_params=pltpu.CompilerParams(
            needs_layout_passes=False, use_tc_tiling_on_sc=False),
    )(_body)
    return f(user.astype(jnp.int32), item.astype(jnp.int32), utab_t, itab_t)


# speedup vs baseline: 17.5775x; 17.5775x over previous
import functools

import jax
import jax.numpy as jnp
from jax import lax
from jax.experimental import pallas as pl
from jax.experimental.pallas import tpu as pltpu
from jax.experimental.pallas import tpu_sc as plsc

_B = 16384
_D = 16
_N = 1000000
_NC = 2
_NS = 16
_NW = _NC * _NS
_BPW = _B // _NW
_K = 16
_TMAX = _N // 128 - 1
_SIDE0 = _TMAX * 128
_SIDEW = 256


def _body(user_hbm, item_hbm, utab_hbm, itab_hbm, uside_hbm, iside_hbm,
          out_hbm, idx_u, idx_i, ring_u, ring_i, side_u, side_i, dots, sem):
    wid = lax.axis_index("s") * _NC + lax.axis_index("c")
    base = wid * _BPW

    pltpu.sync_copy(user_hbm.at[pl.ds(base, _BPW)], idx_u)
    pltpu.sync_copy(item_hbm.at[pl.ds(base, _BPW)], idx_i)
    pltpu.sync_copy(uside_hbm, side_u)
    pltpu.sync_copy(iside_hbm, side_i)

    lane = lax.iota(jnp.int32, _D)
    g_idx = lane // 8
    s_idx = lane % 8

    def batch(b, carry):
        i0 = b * _K
        rv_u = idx_u[pl.ds(i0, _K)]
        rv_i = idx_i[pl.ds(i0, _K)]
        cps = []
        for j in range(_K):
            r_u = jnp.sum(jnp.where(lane == j, rv_u, 0), axis=0)
            t_u = jnp.minimum(r_u // 128, _TMAX)
            cps.append(pltpu.async_copy(
                utab_hbm.at[:, :, pl.ds(t_u * 128, 128)], ring_u.at[j], sem))
            r_i = jnp.sum(jnp.where(lane == j, rv_i, 0), axis=0)
            t_i = jnp.minimum(r_i // 128, _TMAX)
            cps.append(pltpu.async_copy(
                itab_hbm.at[:, :, pl.ds(t_i * 128, 128)], ring_i.at[j], sem))
        for cp in cps:
            cp.wait()
        acc = jnp.zeros((_D,), jnp.float32)
        for j in range(_K):
            r_u = jnp.sum(jnp.where(lane == j, rv_u, 0), axis=0)
            t_u = jnp.minimum(r_u // 128, _TMAX)
            col_u = jnp.zeros((_D,), jnp.int32) + (r_u - t_u * 128)
            cs_u = jnp.zeros((_D,), jnp.int32) + jnp.maximum(r_u - _SIDE0, 0)
            v_main = plsc.load_gather(ring_u.at[j], [g_idx, s_idx, col_u])
            v_side = plsc.load_gather(side_u, [g_idx, s_idx, cs_u])
            vu = jnp.where(r_u < _SIDE0, v_main, v_side)

            r_i = jnp.sum(jnp.where(lane == j, rv_i, 0), axis=0)
            t_i = jnp.minimum(r_i // 128, _TMAX)
            col_i = jnp.zeros((_D,), jnp.int32) + (r_i - t_i * 128)
            cs_i = jnp.zeros((_D,), jnp.int32) + jnp.maximum(r_i - _SIDE0, 0)
            w_main = plsc.load_gather(ring_i.at[j], [g_idx, s_idx, col_i])
            w_side = plsc.load_gather(side_i, [g_idx, s_idx, cs_i])
            vi = jnp.where(r_i < _SIDE0, w_main, w_side)

            acc = jnp.where(lane == j, jnp.sum(vu * vi, axis=0), acc)
        dots[pl.ds(i0, _K)] = acc
        return carry

    lax.fori_loop(0, _BPW // _K, batch, 0)

    def sig(k, carry):
        v = dots[pl.ds(k * _D, _D)]
        dots[pl.ds(k * _D, _D)] = 1.0 / (1.0 + jnp.exp(-v))
        return carry

    lax.fori_loop(0, _BPW // _D, sig, 0)

    pltpu.sync_copy(dots, out_hbm.at[pl.ds(base, _BPW)])


def kernel(user, item, user_table, item_table):
    utab3 = user_table.T.reshape(2, 8, _N)
    itab3 = item_table.T.reshape(2, 8, _N)
    npad = _SIDEW - (_N - _SIDE0)
    uside = jnp.pad(user_table[_SIDE0:].T, ((0, 0), (0, npad))
                    ).reshape(2, 8, _SIDEW)
    iside = jnp.pad(item_table[_SIDE0:].T, ((0, 0), (0, npad))
                    ).reshape(2, 8, _SIDEW)
    mesh = plsc.VectorSubcoreMesh(core_axis_name="c", subcore_axis_name="s")
    f = functools.partial(
        pl.kernel,
        out_type=jax.ShapeDtypeStruct((_B,), jnp.float32),
        mesh=mesh,
        scratch_types=[
            pltpu.VMEM((_BPW,), jnp.int32),
            pltpu.VMEM((_BPW,), jnp.int32),
            pltpu.VMEM((_K, 2, 8, 128), jnp.float32),
            pltpu.VMEM((_K, 2, 8, 128), jnp.float32),
            pltpu.VMEM((2, 8, _SIDEW), jnp.float32),
            pltpu.VMEM((2, 8, _SIDEW), jnp.float32),
            pltpu.VMEM((_BPW,), jnp.float32),
            pltpu.SemaphoreType.DMA,
        ],
        compiler_params=pltpu.CompilerParams(
            needs_layout_passes=False, use_tc_tiling_on_sc=True),
    )(_body)
    return f(user.astype(jnp.int32), item.astype(jnp.int32),
             utab3, itab3, uside, iside)
